# baseline (speedup 1.0000x reference)
import functools

import jax
import jax.numpy as jnp
from jax import lax
from jax.experimental import pallas as pl
from jax.experimental.pallas import tpu as pltpu
from jax.experimental.pallas import tpu_sc as plsc

N = 50000
F = 64
E = 800000
EP = 819200
HALF = 25000
PADH = 25088
SEG = 1568
TAIL = HALF - 15 * SEG
LIN = 6400
BLK = 128
NBLK = LIN // BLK
TE_AGG = EP // 16
NLIN_AGG = TE_AGG // LIN
LIN_A = 5120
NBLK_A = LIN_A // BLK
NLIN_A = TE_AGG // LIN_A
QR = 12500
QPAD = 12544
SEGQ = QPAD // 16
TAILQ = QR - 15 * SEGQ
CBUF = LIN_A + BLK
TE_DEC = EP // 32
NLIN_DEC = TE_DEC // LIN
DEC_LIN0 = 2
RB = 400
NRB = N // RB

_MESH = plsc.VectorSubcoreMesh(core_axis_name="c", subcore_axis_name="s")
_F32 = jnp.float32
_I32 = jnp.int32


def _scatter_indices(dstb, idx2, j, half_lo, s, lanes):
  def grp(q, _):
    d16 = dstb[pl.ds(j * BLK + q * 16, 16)]
    loc = d16 - half_lo
    m = (loc >= 0) & (loc < HALF)
    dummy = HALF + (s % 5) * 16 + lanes
    idx2[j, pl.ds(q * 16, 16)] = jnp.where(m, loc, dummy)
    return 0

  lax.fori_loop(0, BLK // 16, grp, 0)


@functools.partial(
    pl.kernel,
    mesh=_MESH,
    compiler_params=pltpu.CompilerParams(use_tc_tiling_on_sc=False, needs_layout_passes=False),
    out_type=jax.ShapeDtypeStruct((N,), _F32),
    scratch_types=[
        pltpu.VMEM((LIN,), _I32),
        pltpu.VMEM((LIN,), _F32),
        pltpu.VMEM((NBLK, BLK), _I32),
        pltpu.VMEM((SEG,), _F32),
        pltpu.VMEM_SHARED((PADH,), _F32),
        pltpu.SemaphoreType.DMA,
    ],
)
def _degree(dst_hbm, w_hbm, out_hbm, dstb, wb, idx2, zb, acc, sem):
  c = lax.axis_index("c")
  s = lax.axis_index("s")
  lanes = lax.iota(_I32, 16)
  zero16 = jnp.zeros((16,), _F32)

  def zinit(i, _):
    zb[pl.ds(i * 16, 16)] = zero16
    return 0

  lax.fori_loop(0, SEG // 16, zinit, 0)
  pltpu.sync_copy(zb, acc.at[pl.ds(s * SEG, SEG)])
  plsc.subcore_barrier()

  half_lo = c * HALF

  def lin_body(li, _):
    eb = s * TE_AGG + li * LIN
    pltpu.sync_copy(dst_hbm.at[pl.ds(eb, LIN)], dstb)
    pltpu.sync_copy(w_hbm.at[pl.ds(eb, LIN)], wb)

    def blk_fire(j, _):
      _scatter_indices(dstb, idx2, j, half_lo, s, lanes)
      pltpu.async_copy(
          wb.at[pl.ds(j * BLK, BLK)], acc.at[idx2.at[j]], sem, add=True)
      return 0

    lax.fori_loop(0, NBLK, blk_fire, 0)

    def blk_drain(j, _):
      pltpu.make_async_copy(
          wb.at[pl.ds(j * BLK, BLK)], acc.at[idx2.at[j]], sem).wait()
      return 0

    lax.fori_loop(0, NBLK, blk_drain, 0)
    return 0

  lax.fori_loop(0, NLIN_AGG, lin_body, 0)
  plsc.subcore_barrier()

  @pl.when(s < 15)
  def _():
    pltpu.sync_copy(acc.at[pl.ds(s * SEG, SEG)], zb)
    pltpu.sync_copy(zb, out_hbm.at[pl.ds(c * HALF + s * SEG, SEG)])

  @pl.when(s == 15)
  def _():
    pltpu.sync_copy(acc.at[pl.ds(15 * SEG, TAIL)], zb.at[pl.ds(0, TAIL)])
    pltpu.sync_copy(zb.at[pl.ds(0, TAIL)],
                    out_hbm.at[pl.ds(c * HALF + 15 * SEG, TAIL)])


@functools.partial(
    pl.kernel,
    mesh=_MESH,
    compiler_params=pltpu.CompilerParams(use_tc_tiling_on_sc=False, needs_layout_passes=False),
    out_type=jax.ShapeDtypeStruct((N, F), _F32),
    scratch_types=[
        pltpu.VMEM((LIN_A,), _I32),
        pltpu.VMEM((LIN_A,), _I32),
        pltpu.VMEM((LIN_A,), _F32),
        pltpu.VMEM((CBUF,), _I32),
        pltpu.VMEM((CBUF,), _I32),
        pltpu.VMEM((CBUF,), _F32),
        pltpu.VMEM((NBLK_A, BLK), _I32),
        pltpu.VMEM((4, BLK, F), _F32),
        pltpu.VMEM_SHARED((QPAD, F), _F32),
        pltpu.SemaphoreType.DMA((4,)),
        pltpu.SemaphoreType.DMA((4,)),
    ],
)
def _agg(src_hbm, dst_hbm, w_hbm, y_hbm, out_hbm,
         srcb, dstb, wb, csrc, cidx, cw, cidx2, gbuf4, acc_s, gsem, ssem):
  c = lax.axis_index("c")
  s = lax.axis_index("s")
  lanes = lax.iota(_I32, 16)
  zero16 = jnp.zeros((16,), _F32)
  zero16i = jnp.zeros((16,), _I32)
  dummy16 = QR + (s % 2) * 16 + lanes

  def run(acc):
    gb0 = gbuf4.at[0]

    def zrow(i, _):
      for q in range(F // 16):
        gb0[i, pl.ds(q * 16, 16)] = zero16
      return 0

    for p in range(2):
      qlo = (2 * p + c) * QR

      lax.fori_loop(0, BLK, zrow, 0)

      def zcp(k, _):
        pltpu.sync_copy(gb0, acc.at[pl.ds(s * SEGQ + k * BLK, BLK)])
        return 0

      lax.fori_loop(0, SEGQ // BLK, zcp, 0)
      pltpu.sync_copy(gb0.at[pl.ds(0, SEGQ % BLK)],
                      acc.at[pl.ds(s * SEGQ + (SEGQ // BLK) * BLK,
                                   SEGQ % BLK)])
      plsc.subcore_barrier()

      def lin_body(li, _):
        eb = s * TE_AGG + li * LIN_A
        pltpu.sync_copy(src_hbm.at[pl.ds(eb, LIN_A)], srcb)
        pltpu.sync_copy(dst_hbm.at[pl.ds(eb, LIN_A)], dstb)
        pltpu.sync_copy(w_hbm.at[pl.ds(eb, LIN_A)], wb)

        def nfill(k, _):
          csrc[pl.ds(k * 16, 16)] = zero16i
          cidx[pl.ds(k * 16, 16)] = dummy16
          cw[pl.ds(k * 16, 16)] = zero16
          return 0

        lax.fori_loop(0, CBUF // 16, nfill, 0)

        def filt(g, off):
          d16 = dstb[pl.ds(g * 16, 16)]
          loc = d16 - qlo
          m = (loc >= 0) & (loc < QR)
          plsc.store_compressed(cidx.at[pl.ds(off, 16)], loc, mask=m)
          plsc.store_compressed(
              csrc.at[pl.ds(off, 16)], srcb[pl.ds(g * 16, 16)], mask=m)
          plsc.store_compressed(
              cw.at[pl.ds(off, 16)], wb[pl.ds(g * 16, 16)], mask=m)
          return off + plsc.all_reduce_population_count(m)[0]

        off = lax.fori_loop(0, LIN_A // 16, filt, jnp.int32(0))
        nblk = (off + BLK - 1) // BLK

        def repack(k, _):
          @pl.when(k < nblk)
          def _():
            for q in range(BLK // 16):
              cidx2[k, pl.ds(q * 16, 16)] = cidx[pl.ds(k * BLK + q * 16, 16)]
          return 0

        lax.fori_loop(0, NBLK_A, repack, 0)

        for j in range(3):
          @pl.when(j < nblk)
          def _():
            pltpu.async_copy(
                y_hbm.at[csrc.at[pl.ds(j * BLK, BLK)]],
                gbuf4.at[j], gsem.at[j])

        def quad(jj, _):
          for b in range(4):
            j4 = jj * 4 + b
            pb = (b + 3) % 4

            @pl.when(j4 < nblk)
            def _():
              pltpu.make_async_copy(
                  y_hbm.at[csrc.at[pl.ds(j4 * BLK, BLK)]],
                  gbuf4.at[b], gsem.at[b]).wait()

            @pl.when((j4 >= 1) & (j4 - 1 < nblk))
            def _():
              pltpu.make_async_copy(
                  gbuf4.at[pb], acc.at[cidx2.at[j4 - 1]], ssem.at[pb]).wait()

            @pl.when(j4 + 3 < nblk)
            def _():
              pltpu.async_copy(
                  y_hbm.at[csrc.at[pl.ds((j4 + 3) * BLK, BLK)]],
                  gbuf4.at[pb], gsem.at[pb])

            @pl.when(j4 < nblk)
            def _():
              gbuf = gbuf4.at[b]

              def edge_grp(g, _):
                wv = cw[pl.ds(j4 * BLK + g * 16, 16)]
                for i16 in range(16):
                  i = g * 16 + i16
                  ws = wv[i16]
                  for q in range(F // 16):
                    gbuf[i, pl.ds(q * 16, 16)] = (
                        gbuf[i, pl.ds(q * 16, 16)] * ws)
                return 0

              lax.fori_loop(0, BLK // 16, edge_grp, 0)
              pltpu.async_copy(gbuf, acc.at[cidx2.at[j4]], ssem.at[b],
                               add=True)
          return 0

        lax.fori_loop(0, (NBLK_A + 4) // 4, quad, 0)
        return 0

      lax.fori_loop(0, NLIN_A, lin_body, 0)
      plsc.subcore_barrier()

      def wback(k, _):
        r0 = s * SEGQ + k * BLK
        pltpu.sync_copy(acc.at[pl.ds(r0, BLK)], gb0)
        pltpu.sync_copy(gb0, out_hbm.at[pl.ds(qlo + r0, BLK)])
        return 0

      @pl.when(s < 15)
      def _():
        lax.fori_loop(0, SEGQ // BLK, wback, 0)
        r0 = s * SEGQ + (SEGQ // BLK) * BLK
        nrest = SEGQ % BLK
        pltpu.sync_copy(acc.at[pl.ds(r0, nrest)], gb0.at[pl.ds(0, nrest)])
        pltpu.sync_copy(gb0.at[pl.ds(0, nrest)],
                        out_hbm.at[pl.ds(qlo + r0, nrest)])

      @pl.when(s == 15)
      def _():
        lax.fori_loop(0, TAILQ // BLK, wback, 0)
        r0 = s * SEGQ + (TAILQ // BLK) * BLK
        nrest = TAILQ % BLK
        pltpu.sync_copy(acc.at[pl.ds(r0, nrest)], gb0.at[pl.ds(0, nrest)])
        pltpu.sync_copy(gb0.at[pl.ds(0, nrest)],
                        out_hbm.at[pl.ds(qlo + r0, nrest)])
      plsc.subcore_barrier()

  run(acc_s)


@functools.partial(
    pl.kernel,
    mesh=_MESH,
    compiler_params=pltpu.CompilerParams(use_tc_tiling_on_sc=False, needs_layout_passes=False),
    out_type=jax.ShapeDtypeStruct((EP,), _F32),
    scratch_types=[
        pltpu.VMEM((LIN,), _I32),
        pltpu.VMEM((LIN,), _I32),
        pltpu.VMEM((LIN,), _F32),
        pltpu.VMEM((4, BLK, F), _F32),
        pltpu.VMEM((4, BLK, F), _F32),
        pltpu.SemaphoreType.DMA((4,)),
        pltpu.SemaphoreType.DMA((4,)),
    ],
)
def _decode(src_hbm, dst_hbm, z_hbm, p_hbm,
            srcb, dstb, dotb, gsb, gdb, ssem, dsem):
  c = lax.axis_index("c")
  s = lax.axis_index("s")
  lanes = lax.iota(_I32, 16)
  ebase = s * TE_AGG + jnp.where(c == 0, 0, DEC_LIN0 * LIN)
  nlin_c = jnp.where(c == 0, DEC_LIN0, TE_AGG // LIN - DEC_LIN0)

  def start_pair(j):
    b = j % 4
    pltpu.async_copy(
        z_hbm.at[srcb.at[pl.ds(j * BLK, BLK)]], gsb.at[b], ssem.at[b])
    pltpu.async_copy(
        z_hbm.at[dstb.at[pl.ds(j * BLK, BLK)]], gdb.at[b], dsem.at[b])

  def wait_pair(j):
    b = j % 4
    pltpu.make_async_copy(
        z_hbm.at[srcb.at[pl.ds(j * BLK, BLK)]], gsb.at[b], ssem.at[b]).wait()
    pltpu.make_async_copy(
        z_hbm.at[dstb.at[pl.ds(j * BLK, BLK)]], gdb.at[b], dsem.at[b]).wait()

  def lin_body(li, _):
    eb = ebase + li * LIN
    pltpu.sync_copy(src_hbm.at[pl.ds(eb, LIN)], srcb)
    pltpu.sync_copy(dst_hbm.at[pl.ds(eb, LIN)], dstb)
    for j in range(3):
      start_pair(j)

    def quad(jj, _):
      for b in range(4):
        j4 = jj * 4 + b

        @pl.when(j4 < NBLK)
        def _():
          j = jj * 4 + b
          wait_pair(j)

          @pl.when(j4 + 3 < NBLK)
          def _():
            pltpu.async_copy(
                z_hbm.at[srcb.at[pl.ds((j + 3) * BLK, BLK)]],
                gsb.at[(b + 3) % 4], ssem.at[(b + 3) % 4])
            pltpu.async_copy(
                z_hbm.at[dstb.at[pl.ds((j + 3) * BLK, BLK)]],
                gdb.at[(b + 3) % 4], dsem.at[(b + 3) % 4])

          ga = gsb.at[b]
          gb = gdb.at[b]

          def edge_grp(g, _):
            vals = jnp.zeros((16,), _F32)
            for i16 in range(16):
              i = g * 16 + i16
              v = ga[i, pl.ds(0, 16)] * gb[i, pl.ds(0, 16)]
              for q in range(1, F // 16):
                v = v + ga[i, pl.ds(q * 16, 16)] * gb[i, pl.ds(q * 16, 16)]
              vals = jnp.where(lanes == i16, jnp.sum(v), vals)
            dotb[pl.ds(j * BLK + g * 16, 16)] = vals
            return 0

          lax.fori_loop(0, BLK // 16, edge_grp, 0)
      return 0

    lax.fori_loop(0, (NBLK + 3) // 4, quad, 0)

    def sig(k, _):
      d16 = dotb[pl.ds(k * 16, 16)]
      dotb[pl.ds(k * 16, 16)] = 1.0 / (1.0 + jnp.exp(-d16))
      return 0

    lax.fori_loop(0, LIN // 16, sig, 0)
    pltpu.sync_copy(dotb, p_hbm.at[pl.ds(eb, LIN)])
    return 0

  lax.fori_loop(0, nlin_c, lin_body, 0)


def _prep_body(x_ref, w_ref, dg_ref, xw_ref, y_ref, dis_ref, dinv_ref):
  deg = dg_ref[...] + 1.0
  dis = lax.rsqrt(deg)
  dinv = 1.0 / deg
  xw = jnp.dot(x_ref[...], w_ref[...], preferred_element_type=_F32)
  xw_ref[...] = xw
  y_ref[...] = xw * dis
  dis_ref[...] = dis
  dinv_ref[...] = dinv


def _mid_body(agg_ref, xw_ref, dis_ref, dinv_ref, b_ref, w2_ref,
              xw2_ref, y2_ref):
  h = jnp.maximum(
      dis_ref[...] * agg_ref[...] + xw_ref[...] * dinv_ref[...] + b_ref[...],
      0.0)
  xw2 = jnp.dot(h, w2_ref[...], preferred_element_type=_F32)
  xw2_ref[...] = xw2
  y2_ref[...] = xw2 * dis_ref[...]


def _fin_body(agg_ref, xw_ref, dis_ref, dinv_ref, b_ref, x_ref,
              z_ref, enc_ref):
  z = jax.nn.sigmoid(
      dis_ref[...] * agg_ref[...] + xw_ref[...] * dinv_ref[...] + b_ref[...])
  z_ref[...] = z
  enc_ref[...] = z + x_ref[...]


def _row_spec(last):
  return pl.BlockSpec((RB, last), lambda i: (i, 0))


def _rep_spec(shape):
  return pl.BlockSpec(shape, lambda i: (0, 0))


_prep = pl.pallas_call(
    _prep_body,
    grid=(NRB,),
    in_specs=[_row_spec(F), _rep_spec((F, F)), _row_spec(1)],
    out_specs=[_row_spec(F), _row_spec(F), _row_spec(1), _row_spec(1)],
    out_shape=[
        jax.ShapeDtypeStruct((N, F), _F32),
        jax.ShapeDtypeStruct((N, F), _F32),
        jax.ShapeDtypeStruct((N, 1), _F32),
        jax.ShapeDtypeStruct((N, 1), _F32),
    ],
)

_mid = pl.pallas_call(
    _mid_body,
    grid=(NRB,),
    in_specs=[_row_spec(F), _row_spec(F), _row_spec(1), _row_spec(1),
              _rep_spec((1, F)), _rep_spec((F, F))],
    out_specs=[_row_spec(F), _row_spec(F)],
    out_shape=[
        jax.ShapeDtypeStruct((N, F), _F32),
        jax.ShapeDtypeStruct((N, F), _F32),
    ],
)

_fin = pl.pallas_call(
    _fin_body,
    grid=(NRB,),
    in_specs=[_row_spec(F), _row_spec(F), _row_spec(1), _row_spec(1),
              _rep_spec((1, F)), _row_spec(F)],
    out_specs=[_row_spec(F), _row_spec(F)],
    out_shape=[
        jax.ShapeDtypeStruct((N, F), _F32),
        jax.ShapeDtypeStruct((N, F), _F32),
    ],
)


def kernel(node_features, edge_list, edge_attr, W1, b1, W2, b2):
  src = edge_list[0]
  dst = edge_list[1]
  padi = jnp.zeros((EP - E,), _I32)
  srcp = jnp.concatenate([src, padi])
  dstp = jnp.concatenate([dst, padi])
  wp = jnp.concatenate([edge_attr, jnp.zeros((EP - E,), _F32)])

  degraw = _degree(dstp, wp)
  xw1, y1, dis2d, dinv2d = _prep(node_features, W1, degraw[:, None])
  agg1 = _agg(srcp, dstp, wp, y1)
  xw2, y2 = _mid(agg1, xw1, dis2d, dinv2d, b1.reshape(1, F), W2)
  agg2 = _agg(srcp, dstp, wp, y2)
  z, enc = _fin(agg2, xw2, dis2d, dinv2d, b2.reshape(1, F), node_features)
  p = _decode(srcp, dstp, z)
  return enc, p[:E]


# speedup vs baseline: 1.8148x; 1.8148x over previous
import functools

import jax
import jax.numpy as jnp
from jax import lax
from jax.experimental import pallas as pl
from jax.experimental.pallas import tpu as pltpu
from jax.experimental.pallas import tpu_sc as plsc

N = 50000
F = 64
E = 800000
EP = 819200
HALF = 25000
PADH = 25088
SEG = 1568
TAIL = HALF - 15 * SEG
LIN = 6400
BLK = 128
NBLK = LIN // BLK
TE_AGG = EP // 16
NLIN_AGG = TE_AGG // LIN
LIN_A = 5120
NBLK_A = LIN_A // BLK
NLIN_A = TE_AGG // LIN_A
QR = 12500
QPAD = 12544
SEGQ = QPAD // 16
TAILQ = QR - 15 * SEGQ
CBUF = LIN_A + BLK
TE_DEC = EP // 32
NLIN_DEC = TE_DEC // LIN
DEC_LIN0 = 4
RB = 400
NRB = N // RB

_MESH = plsc.VectorSubcoreMesh(core_axis_name="c", subcore_axis_name="s")
_F32 = jnp.float32
_I32 = jnp.int32


def _scatter_indices(dstb, idx2, j, half_lo, s, lanes):
  def grp(q, _):
    d16 = dstb[pl.ds(j * BLK + q * 16, 16)]
    loc = d16 - half_lo
    m = (loc >= 0) & (loc < HALF)
    dummy = HALF + (s % 5) * 16 + lanes
    idx2[j, pl.ds(q * 16, 16)] = jnp.where(m, loc, dummy)
    return 0

  lax.fori_loop(0, BLK // 16, grp, 0)


@functools.partial(
    pl.kernel,
    mesh=_MESH,
    compiler_params=pltpu.CompilerParams(use_tc_tiling_on_sc=False, needs_layout_passes=False),
    out_type=jax.ShapeDtypeStruct((N,), _F32),
    scratch_types=[
        pltpu.VMEM((LIN,), _I32),
        pltpu.VMEM((LIN,), _F32),
        pltpu.VMEM((NBLK, BLK), _I32),
        pltpu.VMEM((SEG,), _F32),
        pltpu.VMEM_SHARED((PADH,), _F32),
        pltpu.SemaphoreType.DMA,
    ],
)
def _degree(dst_hbm, w_hbm, out_hbm, dstb, wb, idx2, zb, acc, sem):
  c = lax.axis_index("c")
  s = lax.axis_index("s")
  lanes = lax.iota(_I32, 16)
  zero16 = jnp.zeros((16,), _F32)

  def zinit(i, _):
    zb[pl.ds(i * 16, 16)] = zero16
    return 0

  lax.fori_loop(0, SEG // 16, zinit, 0)
  pltpu.sync_copy(zb, acc.at[pl.ds(s * SEG, SEG)])
  plsc.subcore_barrier()

  half_lo = c * HALF

  def lin_body(li, _):
    eb = s * TE_AGG + li * LIN
    pltpu.sync_copy(dst_hbm.at[pl.ds(eb, LIN)], dstb)
    pltpu.sync_copy(w_hbm.at[pl.ds(eb, LIN)], wb)

    def blk_fire(j, _):
      _scatter_indices(dstb, idx2, j, half_lo, s, lanes)
      pltpu.async_copy(
          wb.at[pl.ds(j * BLK, BLK)], acc.at[idx2.at[j]], sem, add=True)
      return 0

    lax.fori_loop(0, NBLK, blk_fire, 0)

    def blk_drain(j, _):
      pltpu.make_async_copy(
          wb.at[pl.ds(j * BLK, BLK)], acc.at[idx2.at[j]], sem).wait()
      return 0

    lax.fori_loop(0, NBLK, blk_drain, 0)
    return 0

  lax.fori_loop(0, NLIN_AGG, lin_body, 0)
  plsc.subcore_barrier()

  @pl.when(s < 15)
  def _():
    pltpu.sync_copy(acc.at[pl.ds(s * SEG, SEG)], zb)
    pltpu.sync_copy(zb, out_hbm.at[pl.ds(c * HALF + s * SEG, SEG)])

  @pl.when(s == 15)
  def _():
    pltpu.sync_copy(acc.at[pl.ds(15 * SEG, TAIL)], zb.at[pl.ds(0, TAIL)])
    pltpu.sync_copy(zb.at[pl.ds(0, TAIL)],
                    out_hbm.at[pl.ds(c * HALF + 15 * SEG, TAIL)])


@functools.partial(
    pl.kernel,
    mesh=_MESH,
    compiler_params=pltpu.CompilerParams(use_tc_tiling_on_sc=False, needs_layout_passes=False),
    out_type=jax.ShapeDtypeStruct((N, F), _F32),
    scratch_types=[
        pltpu.VMEM((LIN_A,), _I32),
        pltpu.VMEM((LIN_A,), _I32),
        pltpu.VMEM((LIN_A,), _F32),
        pltpu.VMEM((CBUF,), _I32),
        pltpu.VMEM((CBUF,), _I32),
        pltpu.VMEM((CBUF,), _F32),
        pltpu.VMEM((NBLK_A, BLK), _I32),
        pltpu.VMEM((4, BLK, F), _F32),
        pltpu.VMEM_SHARED((QPAD, F), _F32),
        pltpu.SemaphoreType.DMA((4,)),
        pltpu.SemaphoreType.DMA((4,)),
    ],
)
def _agg(src_hbm, dst_hbm, w_hbm, y_hbm, out_hbm,
         srcb, dstb, wb, csrc, cidx, cw, cidx2, gbuf4, acc_s, gsem, ssem):
  c = lax.axis_index("c")
  s = lax.axis_index("s")
  lanes = lax.iota(_I32, 16)
  zero16 = jnp.zeros((16,), _F32)
  zero16i = jnp.zeros((16,), _I32)
  dummy16 = QR + (s % 2) * 16 + lanes

  def run(acc):
    gb0 = gbuf4.at[0]

    def zrow(i, _):
      for q in range(F // 16):
        gb0[i, pl.ds(q * 16, 16)] = zero16
      return 0

    for p in range(2):
      qlo = (2 * p + c) * QR

      lax.fori_loop(0, BLK, zrow, 0)

      def zcp(k, _):
        pltpu.sync_copy(gb0, acc.at[pl.ds(s * SEGQ + k * BLK, BLK)])
        return 0

      lax.fori_loop(0, SEGQ // BLK, zcp, 0)
      pltpu.sync_copy(gb0.at[pl.ds(0, SEGQ % BLK)],
                      acc.at[pl.ds(s * SEGQ + (SEGQ // BLK) * BLK,
                                   SEGQ % BLK)])
      plsc.subcore_barrier()

      def lin_body(li, _):
        eb = s * TE_AGG + li * LIN_A
        pltpu.sync_copy(src_hbm.at[pl.ds(eb, LIN_A)], srcb)
        pltpu.sync_copy(dst_hbm.at[pl.ds(eb, LIN_A)], dstb)
        pltpu.sync_copy(w_hbm.at[pl.ds(eb, LIN_A)], wb)

        def nfill(k, _):
          csrc[pl.ds(k * 16, 16)] = zero16i
          cidx[pl.ds(k * 16, 16)] = dummy16
          cw[pl.ds(k * 16, 16)] = zero16
          return 0

        lax.fori_loop(0, CBUF // 16, nfill, 0)

        def filt(g, off):
          d16 = dstb[pl.ds(g * 16, 16)]
          loc = d16 - qlo
          m = (loc >= 0) & (loc < QR)
          plsc.store_compressed(cidx.at[pl.ds(off, 16)], loc, mask=m)
          plsc.store_compressed(
              csrc.at[pl.ds(off, 16)], srcb[pl.ds(g * 16, 16)], mask=m)
          plsc.store_compressed(
              cw.at[pl.ds(off, 16)], wb[pl.ds(g * 16, 16)], mask=m)
          return off + plsc.all_reduce_population_count(m)[0]

        off = lax.fori_loop(0, LIN_A // 16, filt, jnp.int32(0))
        nblk = (off + BLK - 1) // BLK

        def repack(k, _):
          @pl.when(k < nblk)
          def _():
            for q in range(BLK // 16):
              cidx2[k, pl.ds(q * 16, 16)] = cidx[pl.ds(k * BLK + q * 16, 16)]
          return 0

        lax.fori_loop(0, NBLK_A, repack, 0)

        for j in range(3):
          @pl.when(j < nblk)
          def _():
            pltpu.async_copy(
                y_hbm.at[csrc.at[pl.ds(j * BLK, BLK)]],
                gbuf4.at[j], gsem.at[j])

        def quad(jj, _):
          for b in range(4):
            j4 = jj * 4 + b
            pb = (b + 3) % 4

            @pl.when(j4 < nblk)
            def _():
              pltpu.make_async_copy(
                  y_hbm.at[csrc.at[pl.ds(j4 * BLK, BLK)]],
                  gbuf4.at[b], gsem.at[b]).wait()

            @pl.when((j4 >= 1) & (j4 - 1 < nblk))
            def _():
              pltpu.make_async_copy(
                  gbuf4.at[pb], acc.at[cidx2.at[j4 - 1]], ssem.at[pb]).wait()

            @pl.when(j4 + 3 < nblk)
            def _():
              pltpu.async_copy(
                  y_hbm.at[csrc.at[pl.ds((j4 + 3) * BLK, BLK)]],
                  gbuf4.at[pb], gsem.at[pb])

            @pl.when(j4 < nblk)
            def _():
              gbuf = gbuf4.at[b]

              def edge_grp(g, _):
                wv = cw[pl.ds(j4 * BLK + g * 16, 16)]
                for i16 in range(16):
                  i = g * 16 + i16
                  ws = wv[i16]
                  for q in range(F // 16):
                    gbuf[i, pl.ds(q * 16, 16)] = (
                        gbuf[i, pl.ds(q * 16, 16)] * ws)
                return 0

              lax.fori_loop(0, BLK // 16, edge_grp, 0)
              pltpu.async_copy(gbuf, acc.at[cidx2.at[j4]], ssem.at[b],
                               add=True)
          return 0

        lax.fori_loop(0, (NBLK_A + 4) // 4, quad, 0)
        return 0

      lax.fori_loop(0, NLIN_A, lin_body, 0)
      plsc.subcore_barrier()

      def wback(k, _):
        r0 = s * SEGQ + k * BLK
        pltpu.sync_copy(acc.at[pl.ds(r0, BLK)], gb0)
        pltpu.sync_copy(gb0, out_hbm.at[pl.ds(qlo + r0, BLK)])
        return 0

      @pl.when(s < 15)
      def _():
        lax.fori_loop(0, SEGQ // BLK, wback, 0)
        r0 = s * SEGQ + (SEGQ // BLK) * BLK
        nrest = SEGQ % BLK
        pltpu.sync_copy(acc.at[pl.ds(r0, nrest)], gb0.at[pl.ds(0, nrest)])
        pltpu.sync_copy(gb0.at[pl.ds(0, nrest)],
                        out_hbm.at[pl.ds(qlo + r0, nrest)])

      @pl.when(s == 15)
      def _():
        lax.fori_loop(0, TAILQ // BLK, wback, 0)
        r0 = s * SEGQ + (TAILQ // BLK) * BLK
        nrest = TAILQ % BLK
        pltpu.sync_copy(acc.at[pl.ds(r0, nrest)], gb0.at[pl.ds(0, nrest)])
        pltpu.sync_copy(gb0.at[pl.ds(0, nrest)],
                        out_hbm.at[pl.ds(qlo + r0, nrest)])
      plsc.subcore_barrier()

  run(acc_s)


@functools.partial(
    pl.kernel,
    mesh=_MESH,
    compiler_params=pltpu.CompilerParams(use_tc_tiling_on_sc=False, needs_layout_passes=False),
    out_type=jax.ShapeDtypeStruct((EP,), _F32),
    scratch_types=[
        pltpu.VMEM((LIN,), _I32),
        pltpu.VMEM((LIN,), _I32),
        pltpu.VMEM((LIN,), _F32),
        pltpu.VMEM((4, BLK, F), _F32),
        pltpu.VMEM((4, BLK, F), _F32),
        pltpu.SemaphoreType.DMA((4,)),
        pltpu.SemaphoreType.DMA((4,)),
    ],
)
def _decode(src_hbm, dst_hbm, z_hbm, p_hbm,
            srcb, dstb, dotb, gsb, gdb, ssem, dsem):
  c = lax.axis_index("c")
  s = lax.axis_index("s")
  lanes = lax.iota(_I32, 16)
  ebase = s * TE_AGG + jnp.where(c == 0, 0, DEC_LIN0 * LIN)
  nlin_c = jnp.where(c == 0, DEC_LIN0, TE_AGG // LIN - DEC_LIN0)

  def start_pair(j):
    b = j % 4
    pltpu.async_copy(
        z_hbm.at[srcb.at[pl.ds(j * BLK, BLK)]], gsb.at[b], ssem.at[b])
    pltpu.async_copy(
        z_hbm.at[dstb.at[pl.ds(j * BLK, BLK)]], gdb.at[b], dsem.at[b])

  def wait_pair(j):
    b = j % 4
    pltpu.make_async_copy(
        z_hbm.at[srcb.at[pl.ds(j * BLK, BLK)]], gsb.at[b], ssem.at[b]).wait()
    pltpu.make_async_copy(
        z_hbm.at[dstb.at[pl.ds(j * BLK, BLK)]], gdb.at[b], dsem.at[b]).wait()

  def lin_body(li, _):
    eb = ebase + li * LIN
    pltpu.sync_copy(src_hbm.at[pl.ds(eb, LIN)], srcb)
    pltpu.sync_copy(dst_hbm.at[pl.ds(eb, LIN)], dstb)
    for j in range(3):
      start_pair(j)

    def quad(jj, _):
      for b in range(4):
        j4 = jj * 4 + b

        @pl.when(j4 < NBLK)
        def _():
          j = jj * 4 + b
          wait_pair(j)

          @pl.when(j4 + 3 < NBLK)
          def _():
            pltpu.async_copy(
                z_hbm.at[srcb.at[pl.ds((j + 3) * BLK, BLK)]],
                gsb.at[(b + 3) % 4], ssem.at[(b + 3) % 4])
            pltpu.async_copy(
                z_hbm.at[dstb.at[pl.ds((j + 3) * BLK, BLK)]],
                gdb.at[(b + 3) % 4], dsem.at[(b + 3) % 4])

          ga = gsb.at[b]
          gb = gdb.at[b]

          def edge_grp(g, _):
            vals = jnp.zeros((16,), _F32)
            for i16 in range(16):
              i = g * 16 + i16
              v = ga[i, pl.ds(0, 16)] * gb[i, pl.ds(0, 16)]
              for q in range(1, F // 16):
                v = v + ga[i, pl.ds(q * 16, 16)] * gb[i, pl.ds(q * 16, 16)]
              vals = jnp.where(lanes == i16, jnp.sum(v), vals)
            dotb[pl.ds(j * BLK + g * 16, 16)] = vals
            return 0

          lax.fori_loop(0, BLK // 16, edge_grp, 0)
      return 0

    lax.fori_loop(0, (NBLK + 3) // 4, quad, 0)

    def sig(k, _):
      d16 = dotb[pl.ds(k * 16, 16)]
      dotb[pl.ds(k * 16, 16)] = 1.0 / (1.0 + jnp.exp(-d16))
      return 0

    lax.fori_loop(0, LIN // 16, sig, 0)
    pltpu.sync_copy(dotb, p_hbm.at[pl.ds(eb, LIN)])
    return 0

  lax.fori_loop(0, nlin_c, lin_body, 0)


def _prep_body(x_ref, w_ref, dg_ref, xw_ref, y_ref, dis_ref, dinv_ref):
  deg = dg_ref[...] + 1.0
  dis = lax.rsqrt(deg)
  dinv = 1.0 / deg
  xw = jnp.dot(x_ref[...], w_ref[...], preferred_element_type=_F32)
  xw_ref[...] = xw
  y_ref[...] = xw * dis
  dis_ref[...] = dis
  dinv_ref[...] = dinv


def _mid_body(agg_ref, xw_ref, dis_ref, dinv_ref, b_ref, w2_ref,
              xw2_ref, y2_ref):
  h = jnp.maximum(
      dis_ref[...] * agg_ref[...] + xw_ref[...] * dinv_ref[...] + b_ref[...],
      0.0)
  xw2 = jnp.dot(h, w2_ref[...], preferred_element_type=_F32)
  xw2_ref[...] = xw2
  y2_ref[...] = xw2 * dis_ref[...]


def _fin_body(agg_ref, xw_ref, dis_ref, dinv_ref, b_ref, x_ref,
              z_ref, enc_ref):
  z = jax.nn.sigmoid(
      dis_ref[...] * agg_ref[...] + xw_ref[...] * dinv_ref[...] + b_ref[...])
  z_ref[...] = z
  enc_ref[...] = z + x_ref[...]


def _row_spec(last):
  return pl.BlockSpec((RB, last), lambda i: (i, 0))


def _rep_spec(shape):
  return pl.BlockSpec(shape, lambda i: (0, 0))


_prep = pl.pallas_call(
    _prep_body,
    grid=(NRB,),
    in_specs=[_row_spec(F), _rep_spec((F, F)), _row_spec(1)],
    out_specs=[_row_spec(F), _row_spec(F), _row_spec(1), _row_spec(1)],
    out_shape=[
        jax.ShapeDtypeStruct((N, F), _F32),
        jax.ShapeDtypeStruct((N, F), _F32),
        jax.ShapeDtypeStruct((N, 1), _F32),
        jax.ShapeDtypeStruct((N, 1), _F32),
    ],
)

_mid = pl.pallas_call(
    _mid_body,
    grid=(NRB,),
    in_specs=[_row_spec(F), _row_spec(F), _row_spec(1), _row_spec(1),
              _rep_spec((1, F)), _rep_spec((F, F))],
    out_specs=[_row_spec(F), _row_spec(F)],
    out_shape=[
        jax.ShapeDtypeStruct((N, F), _F32),
        jax.ShapeDtypeStruct((N, F), _F32),
    ],
)

_fin = pl.pallas_call(
    _fin_body,
    grid=(NRB,),
    in_specs=[_row_spec(F), _row_spec(F), _row_spec(1), _row_spec(1),
              _rep_spec((1, F)), _row_spec(F)],
    out_specs=[_row_spec(F), _row_spec(F)],
    out_shape=[
        jax.ShapeDtypeStruct((N, F), _F32),
        jax.ShapeDtypeStruct((N, F), _F32),
    ],
)


def kernel(node_features, edge_list, edge_attr, W1, b1, W2, b2):
  src = edge_list[0]
  dst = edge_list[1]
  padi = jnp.arange(EP - E, dtype=_I32) % N
  srcp = jnp.concatenate([src, padi])
  dstp = jnp.concatenate([dst, padi])
  wp = jnp.concatenate([edge_attr, jnp.zeros((EP - E,), _F32)])

  degraw = _degree(dstp, wp)
  xw1, y1, dis2d, dinv2d = _prep(node_features, W1, degraw[:, None])
  agg1 = _agg(srcp, dstp, wp, y1)
  xw2, y2 = _mid(agg1, xw1, dis2d, dinv2d, b1.reshape(1, F), W2)
  agg2 = _agg(srcp, dstp, wp, y2)
  z, enc = _fin(agg2, xw2, dis2d, dinv2d, b2.reshape(1, F), node_features)
  p = _decode(srcp, dstp, z)
  return enc, p[:E]
